# baseline (speedup 1.0000x reference)
import functools

import jax
import jax.numpy as jnp
from jax import lax
from jax.experimental import pallas as pl
from jax.experimental.pallas import tpu as pltpu
from jax.experimental.pallas import tpu_sc as plsc

NUM_EMBEDDINGS = 1000000
EMBEDDING_DIM = 32
SHARED_DIM = 4
TABLE_DIM = 28
BATCH = 16384

_info = plsc.get_sparse_core_info()
_NC, _NS = _info.num_cores, _info.num_subcores
_NW = _NC * _NS
_BPW = BATCH // _NW
_CHUNK = 128
_NCHUNK = _BPW // _CHUNK
_L = 16


@functools.partial(
    pl.kernel,
    mesh=plsc.VectorSubcoreMesh(core_axis_name="c", subcore_axis_name="s"),
    out_type=jax.ShapeDtypeStruct((BATCH, EMBEDDING_DIM), jnp.float32),
    compiler_params=pltpu.CompilerParams(
        use_tc_tiling_on_sc=False, needs_layout_passes=False
    ),
    scratch_types=[
        pltpu.VMEM((_BPW,), jnp.int32),
        pltpu.VMEM((_BPW, TABLE_DIM), jnp.float32),
        pltpu.VMEM((_BPW, EMBEDDING_DIM), jnp.float32),
        pltpu.VMEM((_L,), jnp.float32),
        pltpu.SemaphoreType.DMA,
    ],
)
def _embed_lookup(table_hbm, idx_hbm, pat_hbm, out_hbm,
                  idx_v, rows_v, out_v, pat_v, sem):
    wid = lax.axis_index("s") * _NC + lax.axis_index("c")
    base = wid * _BPW
    pltpu.sync_copy(idx_hbm.at[pl.ds(base, _BPW)], idx_v)
    copies = []
    for c in range(_NCHUNK):
        copies.append(
            pltpu.async_copy(
                table_hbm.at[idx_v.at[pl.ds(c * _CHUNK, _CHUNK)]],
                rows_v.at[pl.ds(c * _CHUNK, _CHUNK)],
                sem,
            )
        )
    pltpu.sync_copy(pat_hbm, pat_v)
    for cp in copies:
        cp.wait()

    lanes = lax.iota(jnp.int32, _L)
    upper_cols = jnp.minimum(lanes + _L, TABLE_DIM - 1)
    is_table_col = lanes < (TABLE_DIM - _L)
    pat = pat_v[...]

    def body(i, _):
        v1 = rows_v[i, pl.ds(0, _L)]
        row_ids = jnp.full((_L,), i, jnp.int32)
        v2g = plsc.load_gather(rows_v, [row_ids, upper_cols])
        v2 = jnp.where(is_table_col, v2g, pat)
        out_v[i, pl.ds(0, _L)] = v1
        out_v[i, pl.ds(_L, _L)] = v2
        return _

    lax.fori_loop(0, _BPW, body, None)
    pltpu.sync_copy(out_v, out_hbm.at[pl.ds(base, _BPW)])


def kernel(x, embed_table, shared_embed):
    idx = x.astype(jnp.int32)
    pat16 = jnp.concatenate(
        [jnp.zeros((_L - SHARED_DIM,), jnp.float32),
         shared_embed.reshape(SHARED_DIM).astype(jnp.float32)]
    )
    out = _embed_lookup(embed_table, idx, pat16)
    return out.reshape(BATCH, 1, EMBEDDING_DIM)
